# Initial kernel scaffold; baseline (speedup 1.0000x reference)
import jax
import jax.numpy as jnp
from jax.experimental import pallas as pl


def kernel(embedding, nuc_w, nuc_b, conv0_w_node, conv0_node_b, conv0_w_edge, conv0_scale, conv0_shift, conv0_ln_gamma, conv0_ln_beta, conv1_w_node, conv1_node_b, conv1_w_edge, conv1_scale, conv1_shift, conv1_ln_gamma, conv1_ln_beta, conv2_w_node, conv2_node_b, conv2_w_edge, conv2_scale, conv2_shift, conv2_ln_gamma, conv2_ln_beta, conv3_w_node, conv3_node_b, conv3_w_edge, conv3_scale, conv3_shift, conv3_ln_gamma, conv3_ln_beta, conv4_w_node, conv4_node_b, conv4_w_edge, conv4_scale, conv4_shift, conv4_ln_gamma, conv4_ln_beta, conv5_w_node, conv5_node_b, conv5_w_edge, conv5_scale, conv5_shift, conv5_ln_gamma, conv5_ln_beta, conv_to_fc_w, conv_to_fc_b, fc0_w, fc0_b, fc1_w, fc1_b, fc2_w, fc2_b, out_w, out_b, atomic_numbers, positions, batch, neighbors_index):
    raise NotImplementedError("write your pallas kernel here")


# speedup vs baseline: 4.7848x; 4.7848x over previous
import functools

import jax
import jax.numpy as jnp
from jax import lax
from jax.experimental import pallas as pl
from jax.experimental.pallas import tpu as pltpu

_VMEM_LIMIT = 48 * 1024 * 1024
_EDGE_TILE = 4096


def _round_up(x, m):
    return ((x + m - 1) // m) * m


def _softplus(x):
    return jnp.maximum(x, 0.0) + jnp.log1p(jnp.exp(-jnp.abs(x)))


def _edge_step(dst_ref, src_ref, d_ref, we_ref, sc_ref, sh_ref, h_scr,
               acc_ref, *, n_nodes, feat, num_g, off_step, pad_off, coeff):
    two_f = 2 * feat
    row = lax.broadcasted_iota(jnp.int32, (n_nodes, 1), 0)
    ohd = (row == dst_ref[0]).astype(jnp.float32)
    ohs = (row == src_ref[0]).astype(jnp.float32)
    gi = lax.broadcasted_iota(jnp.int32, (we_ref.shape[0], 1), 0
                              ).astype(jnp.float32)
    off = jnp.where(gi < num_g, gi * off_step, pad_off)
    u = d_ref[0] - off
    ef = jnp.exp(coeff * (u * u)).astype(jnp.bfloat16)
    dims = (((0,), (0,)), ((), ()))
    z = (lax.dot_general(ohd, h_scr[:, :two_f], dims,
                         preferred_element_type=jnp.float32,
                         precision=lax.Precision.HIGHEST)
         + lax.dot_general(ohs, h_scr[:, two_f:], dims,
                           preferred_element_type=jnp.float32,
                           precision=lax.Precision.HIGHEST)
         + lax.dot_general(ef, we_ref[...], dims,
                           preferred_element_type=jnp.float32))
    z = z * sc_ref[...] + sh_ref[...]
    gate = pl.reciprocal(1.0 + jnp.exp(-z[:, :feat]), approx=True)
    val = _softplus(z[:, feat:])
    msg = (gate * val).astype(jnp.bfloat16)
    acc_ref[...] += jnp.dot(ohd.astype(jnp.bfloat16), msg,
                            preferred_element_type=jnp.float32)


def _finalize(part0, part1, x_prev, gamma, beta):
    a = part0 + part1
    mean = jnp.mean(a, axis=-1, keepdims=True)
    var = jnp.mean((a - mean) ** 2, axis=-1, keepdims=True)
    ln = (a - mean) * lax.rsqrt(var + 1e-5) * gamma + beta
    return _softplus(ln + x_prev)


def _conv0_kernel(dst_ref, src_ref, d_ref, embx_ref, nucw_ref, nucb_ref,
                  wn_ref, nb_ref, we_ref, sc_ref, sh_ref, part_ref, xout_ref,
                  h_scr, acc_ref, *, tpc, n_nodes, feat, num_g, off_step,
                  pad_off, coeff):
    t = pl.program_id(1)

    @pl.when(t == 0)
    def _():
        x = jnp.dot(embx_ref[...], nucw_ref[...],
                    preferred_element_type=jnp.float32) + nucb_ref[...]
        xout_ref[0] = x
        h_scr[...] = jnp.dot(x.astype(jnp.bfloat16), wn_ref[...],
                             preferred_element_type=jnp.float32) + nb_ref[...]
        acc_ref[...] = jnp.zeros_like(acc_ref)

    _edge_step(dst_ref, src_ref, d_ref, we_ref, sc_ref, sh_ref, h_scr,
               acc_ref, n_nodes=n_nodes, feat=feat, num_g=num_g,
               off_step=off_step, pad_off=pad_off, coeff=coeff)

    @pl.when(t == tpc - 1)
    def _():
        part_ref[0] = acc_ref[...]


def _conv_kernel(dst_ref, src_ref, d_ref, pprev_ref, xprev_ref, g_ref, b_ref,
                 wn_ref, nb_ref, we_ref, sc_ref, sh_ref, part_ref, xout_ref,
                 h_scr, acc_ref, *, tpc, n_nodes, feat, num_g, off_step,
                 pad_off, coeff):
    t = pl.program_id(1)

    @pl.when(t == 0)
    def _():
        x = _finalize(pprev_ref[0], pprev_ref[1], xprev_ref[0],
                      g_ref[...], b_ref[...])
        xout_ref[0] = x
        h_scr[...] = jnp.dot(x.astype(jnp.bfloat16), wn_ref[...],
                             preferred_element_type=jnp.float32) + nb_ref[...]
        acc_ref[...] = jnp.zeros_like(acc_ref)

    _edge_step(dst_ref, src_ref, d_ref, we_ref, sc_ref, sh_ref, h_scr,
               acc_ref, n_nodes=n_nodes, feat=feat, num_g=num_g,
               off_step=off_step, pad_off=pad_off, coeff=coeff)

    @pl.when(t == tpc - 1)
    def _():
        part_ref[0] = acc_ref[...]


def _head_kernel(*refs, num_hidden):
    pprev_ref, xprev_ref, g_ref, b_ref, pool_ref = refs[:5]
    x = _finalize(pprev_ref[0], pprev_ref[1], xprev_ref[0],
                  g_ref[...], b_ref[...])
    h = jnp.dot(pool_ref[...], x, preferred_element_type=jnp.float32)
    idx = 5
    for _ in range(num_hidden):
        w = refs[idx][...]
        b = refs[idx + 1][...]
        idx += 2
        h = _softplus(jnp.dot(h.astype(jnp.bfloat16), w,
                              preferred_element_type=jnp.float32) + b)
    w = refs[idx][...]
    b = refs[idx + 1][...]
    o_ref = refs[idx + 2]
    o_ref[...] = jnp.dot(h.astype(jnp.bfloat16), w,
                         preferred_element_type=jnp.float32) + b


def _conv_call(dstr, srcr, dr, prev, wn, nb, we, sc, sh, *, tpc, tile,
               n_nodes, feat, num_g, off_step, pad_off, coeff):
    two_f = 2 * feat
    n_tiles = dstr.shape[0]
    eim = lambda c, t: (c * tpc + t, 0, 0)
    cim2 = lambda c, t: (0, 0)
    cim3 = lambda c, t: (0, 0, 0)
    edge_specs = [pl.BlockSpec((1, 1, tile), eim) for _ in range(3)]
    if prev[0] is None:
        embx, nucw, nucb = prev[1], prev[2], prev[3]
        body = _conv0_kernel
        extra_in = [embx.astype(jnp.bfloat16), nucw.astype(jnp.bfloat16),
                    nucb.reshape(1, feat)]
        extra_specs = [
            pl.BlockSpec(embx.shape, cim2),
            pl.BlockSpec(nucw.shape, cim2),
            pl.BlockSpec((1, feat), cim2),
        ]
    else:
        part_prev, x_prev, lg, lb = prev
        body = _conv_kernel
        extra_in = [part_prev, x_prev, lg.reshape(1, feat), lb.reshape(1, feat)]
        extra_specs = [
            pl.BlockSpec((2, n_nodes, feat), cim3),
            pl.BlockSpec((1, n_nodes, feat), cim3),
            pl.BlockSpec((1, feat), cim2),
            pl.BlockSpec((1, feat), cim2),
        ]
    e_pad = n_tiles * tile
    cost = pl.CostEstimate(
        flops=int(2 * e_pad * two_f * (2 * n_nodes + we.shape[0] + feat)),
        transcendentals=int(e_pad * (we.shape[0] + 2 * feat)),
        bytes_accessed=int(e_pad * 12 + 4 * n_nodes * (two_f + 4 * feat)))
    return pl.pallas_call(
        functools.partial(body, tpc=tpc, n_nodes=n_nodes, feat=feat,
                          num_g=num_g, off_step=off_step, pad_off=pad_off,
                          coeff=coeff),
        out_shape=(jax.ShapeDtypeStruct((2, n_nodes, feat), jnp.float32),
                   jax.ShapeDtypeStruct((2, n_nodes, feat), jnp.float32)),
        grid=(2, tpc),
        in_specs=edge_specs + extra_specs + [
            pl.BlockSpec((feat, 2 * two_f), cim2),
            pl.BlockSpec((1, 2 * two_f), cim2),
            pl.BlockSpec(we.shape, cim2),
            pl.BlockSpec((1, two_f), cim2),
            pl.BlockSpec((1, two_f), cim2),
        ],
        out_specs=(pl.BlockSpec((1, n_nodes, feat), lambda c, t: (c, 0, 0)),
                   pl.BlockSpec((1, n_nodes, feat), lambda c, t: (c, 0, 0))),
        scratch_shapes=[pltpu.VMEM((n_nodes, 2 * two_f), jnp.float32),
                        pltpu.VMEM((n_nodes, feat), jnp.float32)],
        compiler_params=pltpu.CompilerParams(
            dimension_semantics=("parallel", "arbitrary"),
            vmem_limit_bytes=_VMEM_LIMIT),
        cost_estimate=cost,
    )(dstr, srcr, dr, *extra_in, wn.astype(jnp.bfloat16),
      nb.reshape(1, 2 * two_f), we.astype(jnp.bfloat16),
      sc.reshape(1, two_f), sh.reshape(1, two_f))


def _head_call(part_prev, x_prev, lg, lb, pool, hidden, w_out, b_out, *,
               n_nodes, feat):
    n_graphs = pool.shape[0]
    n_targets = w_out.shape[1]
    cim2 = lambda: (0, 0)
    inputs = [part_prev, x_prev, lg.reshape(1, feat), lb.reshape(1, feat),
              pool]
    in_specs = [
        pl.BlockSpec((2, n_nodes, feat), lambda: (0, 0, 0)),
        pl.BlockSpec((1, n_nodes, feat), lambda: (0, 0, 0)),
        pl.BlockSpec((1, feat), cim2),
        pl.BlockSpec((1, feat), cim2),
        pl.BlockSpec(pool.shape, cim2),
    ]
    for w, b in hidden:
        inputs += [w.astype(jnp.bfloat16), b.reshape(1, -1)]
        in_specs += [pl.BlockSpec(w.shape, cim2),
                     pl.BlockSpec((1, b.shape[0]), cim2)]
    inputs += [w_out.astype(jnp.bfloat16), b_out.reshape(1, n_targets)]
    in_specs += [pl.BlockSpec(w_out.shape, cim2),
                 pl.BlockSpec((1, n_targets), cim2)]
    return pl.pallas_call(
        functools.partial(_head_kernel, num_hidden=len(hidden)),
        out_shape=jax.ShapeDtypeStruct((n_graphs, n_targets), jnp.float32),
        in_specs=in_specs,
        out_specs=pl.BlockSpec((n_graphs, n_targets), cim2),
        compiler_params=pltpu.CompilerParams(vmem_limit_bytes=_VMEM_LIMIT),
    )(*inputs)


def kernel(embedding, nuc_w, nuc_b,
           conv0_w_node, conv0_node_b, conv0_w_edge, conv0_scale, conv0_shift, conv0_ln_gamma, conv0_ln_beta,
           conv1_w_node, conv1_node_b, conv1_w_edge, conv1_scale, conv1_shift, conv1_ln_gamma, conv1_ln_beta,
           conv2_w_node, conv2_node_b, conv2_w_edge, conv2_scale, conv2_shift, conv2_ln_gamma, conv2_ln_beta,
           conv3_w_node, conv3_node_b, conv3_w_edge, conv3_scale, conv3_shift, conv3_ln_gamma, conv3_ln_beta,
           conv4_w_node, conv4_node_b, conv4_w_edge, conv4_scale, conv4_shift, conv4_ln_gamma, conv4_ln_beta,
           conv5_w_node, conv5_node_b, conv5_w_edge, conv5_scale, conv5_shift, conv5_ln_gamma, conv5_ln_beta,
           conv_to_fc_w, conv_to_fc_b,
           fc0_w, fc0_b, fc1_w, fc1_b, fc2_w, fc2_b,
           out_w, out_b,
           atomic_numbers, positions, batch, neighbors_index):
    num_graphs = 2
    num_g = 100
    cutoff = 6.0
    n_nodes = atomic_numbers.shape[0]
    feat = nuc_w.shape[1]
    two_f = 2 * feat

    convs = [
        (conv0_w_node, conv0_node_b, conv0_w_edge, conv0_scale, conv0_shift, conv0_ln_gamma, conv0_ln_beta),
        (conv1_w_node, conv1_node_b, conv1_w_edge, conv1_scale, conv1_shift, conv1_ln_gamma, conv1_ln_beta),
        (conv2_w_node, conv2_node_b, conv2_w_edge, conv2_scale, conv2_shift, conv2_ln_gamma, conv2_ln_beta),
        (conv3_w_node, conv3_node_b, conv3_w_edge, conv3_scale, conv3_shift, conv3_ln_gamma, conv3_ln_beta),
        (conv4_w_node, conv4_node_b, conv4_w_edge, conv4_scale, conv4_shift, conv4_ln_gamma, conv4_ln_beta),
        (conv5_w_node, conv5_node_b, conv5_w_edge, conv5_scale, conv5_shift, conv5_ln_gamma, conv5_ln_beta),
    ]

    src = neighbors_index[0]
    dst = neighbors_index[1]
    n_edges = src.shape[0]

    sigma = cutoff / (num_g - 1)
    coeff = -0.5 / float(sigma ** 2)
    d = jnp.linalg.norm(positions[src] - positions[dst],
                        axis=-1).astype(jnp.float32)
    off_step = sigma
    pad_off = 1e3

    tile = _EDGE_TILE
    n_tiles = _round_up(_round_up(max(n_edges, 1), tile) // tile, 2)
    e_pad = n_tiles * tile
    if e_pad != n_edges:
        p = e_pad - n_edges
        src = jnp.concatenate([src, jnp.zeros((p,), src.dtype)])
        dst = jnp.concatenate([dst, jnp.full((p,), -1, dst.dtype)])
        d = jnp.concatenate([d, jnp.zeros((p,), d.dtype)])
    tpc = n_tiles // 2
    dstr = dst.reshape(n_tiles, 1, tile)
    srcr = src.reshape(n_tiles, 1, tile)
    dr = d.reshape(n_tiles, 1, tile)

    embx = embedding[atomic_numbers - 1]

    onehot = (batch[None, :] == jnp.arange(num_graphs,
                                           dtype=batch.dtype)[:, None]
              ).astype(jnp.float32)
    pool = onehot / jnp.maximum(onehot.sum(axis=1, keepdims=True), 1.0)

    prev = (None, embx, nuc_w, nuc_b)
    for k, (wn, nb, we, sc, sh, lg, lb) in enumerate(convs):
        part, xout = _conv_call(
            dstr, srcr, dr, prev, wn, nb, we, sc, sh, tpc=tpc, tile=tile,
            n_nodes=n_nodes, feat=feat, num_g=num_g, off_step=off_step,
            pad_off=pad_off, coeff=coeff)
        prev = (part, xout[:1], lg, lb)

    hidden = [(conv_to_fc_w, conv_to_fc_b), (fc0_w, fc0_b),
              (fc1_w, fc1_b), (fc2_w, fc2_b)]
    part, x_prev, lg, lb = prev
    return _head_call(part, x_prev, lg, lb, pool, hidden, out_w, out_b,
                      n_nodes=n_nodes, feat=feat)


# speedup vs baseline: 6.7130x; 1.4030x over previous
import functools

import jax
import jax.numpy as jnp
from jax import lax
from jax.experimental import pallas as pl
from jax.experimental.pallas import tpu as pltpu

_VMEM_LIMIT = 48 * 1024 * 1024
_EDGE_TILE = 4096


def _round_up(x, m):
    return ((x + m - 1) // m) * m


def _softplus(x):
    return jnp.maximum(x, 0.0) + jnp.log1p(jnp.exp(-jnp.abs(x)))


def _edge_step(dst_ref, src_ref, pt_ref, we_ref, sc_ref, sh_ref, h_scr,
               acc_ref, *, n_nodes, feat, num_g, off_step, pad_off, coeff):
    two_f = 2 * feat
    row = lax.broadcasted_iota(jnp.int32, (n_nodes, 1), 0)
    ohd = (row == dst_ref[0]).astype(jnp.float32)
    ohs = (row == src_ref[0]).astype(jnp.float32)
    diff = (jnp.dot(pt_ref[...], ohd, preferred_element_type=jnp.float32,
                    precision=lax.Precision.HIGHEST)
            - jnp.dot(pt_ref[...], ohs, preferred_element_type=jnp.float32,
                      precision=lax.Precision.HIGHEST))
    d = jnp.sqrt(jnp.sum(diff * diff, axis=0, keepdims=True))
    gi = lax.broadcasted_iota(jnp.int32, (we_ref.shape[0], 1), 0
                              ).astype(jnp.float32)
    off = jnp.where(gi < num_g, gi * off_step, pad_off)
    u = d - off
    ef = jnp.exp(coeff * (u * u)).astype(jnp.bfloat16)
    dims = (((0,), (0,)), ((), ()))
    z = (lax.dot_general(ohd, h_scr[:, :two_f], dims,
                         preferred_element_type=jnp.float32,
                         precision=lax.Precision.HIGHEST)
         + lax.dot_general(ohs, h_scr[:, two_f:], dims,
                           preferred_element_type=jnp.float32,
                           precision=lax.Precision.HIGHEST)
         + lax.dot_general(ef, we_ref[...], dims,
                           preferred_element_type=jnp.float32))
    z = z * sc_ref[...] + sh_ref[...]
    gate = pl.reciprocal(1.0 + jnp.exp(-z[:, :feat]), approx=True)
    val = _softplus(z[:, feat:])
    msg = (gate * val).astype(jnp.bfloat16)
    acc_ref[...] += jnp.dot(ohd.astype(jnp.bfloat16), msg,
                            preferred_element_type=jnp.float32)


def _finalize(part0, part1, x_prev, gamma, beta):
    a = part0 + part1
    mean = jnp.mean(a, axis=-1, keepdims=True)
    var = jnp.mean((a - mean) ** 2, axis=-1, keepdims=True)
    ln = (a - mean) * lax.rsqrt(var + 1e-5) * gamma + beta
    return _softplus(ln + x_prev)


def _conv0_kernel(dst_ref, src_ref, pt_ref, embx_ref, nucw_ref, nucb_ref,
                  wn_ref, nb_ref, we_ref, sc_ref, sh_ref, part_ref, xout_ref,
                  h_scr, acc_ref, *, tpc, n_nodes, feat, num_g, off_step,
                  pad_off, coeff):
    t = pl.program_id(1)

    @pl.when(t == 0)
    def _():
        x = jnp.dot(embx_ref[...], nucw_ref[...],
                    preferred_element_type=jnp.float32) + nucb_ref[...]
        xout_ref[0] = x
        h_scr[...] = jnp.dot(x.astype(jnp.bfloat16), wn_ref[...],
                             preferred_element_type=jnp.float32) + nb_ref[...]
        acc_ref[...] = jnp.zeros_like(acc_ref)

    _edge_step(dst_ref, src_ref, pt_ref, we_ref, sc_ref, sh_ref, h_scr,
               acc_ref, n_nodes=n_nodes, feat=feat, num_g=num_g,
               off_step=off_step, pad_off=pad_off, coeff=coeff)

    @pl.when(t == tpc - 1)
    def _():
        part_ref[0] = acc_ref[...]


def _conv_kernel(dst_ref, src_ref, pt_ref, pprev_ref, xprev_ref, g_ref, b_ref,
                 wn_ref, nb_ref, we_ref, sc_ref, sh_ref, part_ref, xout_ref,
                 h_scr, acc_ref, *, tpc, n_nodes, feat, num_g, off_step,
                 pad_off, coeff):
    t = pl.program_id(1)

    @pl.when(t == 0)
    def _():
        x = _finalize(pprev_ref[0], pprev_ref[1], xprev_ref[0],
                      g_ref[...], b_ref[...])
        xout_ref[0] = x
        h_scr[...] = jnp.dot(x.astype(jnp.bfloat16), wn_ref[...],
                             preferred_element_type=jnp.float32) + nb_ref[...]
        acc_ref[...] = jnp.zeros_like(acc_ref)

    _edge_step(dst_ref, src_ref, pt_ref, we_ref, sc_ref, sh_ref, h_scr,
               acc_ref, n_nodes=n_nodes, feat=feat, num_g=num_g,
               off_step=off_step, pad_off=pad_off, coeff=coeff)

    @pl.when(t == tpc - 1)
    def _():
        part_ref[0] = acc_ref[...]


def _head_kernel(*refs, num_hidden):
    pprev_ref, xprev_ref, g_ref, b_ref, pool_ref = refs[:5]
    x = _finalize(pprev_ref[0], pprev_ref[1], xprev_ref[0],
                  g_ref[...], b_ref[...])
    h = jnp.dot(pool_ref[...], x, preferred_element_type=jnp.float32)
    idx = 5
    for _ in range(num_hidden):
        w = refs[idx][...]
        b = refs[idx + 1][...]
        idx += 2
        h = _softplus(jnp.dot(h.astype(jnp.bfloat16), w,
                              preferred_element_type=jnp.float32) + b)
    w = refs[idx][...]
    b = refs[idx + 1][...]
    o_ref = refs[idx + 2]
    o_ref[...] = jnp.dot(h.astype(jnp.bfloat16), w,
                         preferred_element_type=jnp.float32) + b


def _conv_call(dstr, srcr, pt, prev, wn, nb, we, sc, sh, *, tpc, tile,
               n_nodes, feat, num_g, off_step, pad_off, coeff):
    two_f = 2 * feat
    n_tiles = dstr.shape[0]
    eim = lambda c, t: (c * tpc + t, 0, 0)
    cim2 = lambda c, t: (0, 0)
    cim3 = lambda c, t: (0, 0, 0)
    edge_specs = [pl.BlockSpec((1, 1, tile), eim) for _ in range(2)]
    edge_specs.append(pl.BlockSpec(pt.shape, cim2))
    if prev[0] is None:
        embx, nucw, nucb = prev[1], prev[2], prev[3]
        body = _conv0_kernel
        extra_in = [embx.astype(jnp.bfloat16), nucw.astype(jnp.bfloat16),
                    nucb.reshape(1, feat)]
        extra_specs = [
            pl.BlockSpec(embx.shape, cim2),
            pl.BlockSpec(nucw.shape, cim2),
            pl.BlockSpec((1, feat), cim2),
        ]
    else:
        part_prev, x_prev, lg, lb = prev
        body = _conv_kernel
        extra_in = [part_prev, x_prev, lg.reshape(1, feat), lb.reshape(1, feat)]
        extra_specs = [
            pl.BlockSpec((2, n_nodes, feat), cim3),
            pl.BlockSpec((1, n_nodes, feat), cim3),
            pl.BlockSpec((1, feat), cim2),
            pl.BlockSpec((1, feat), cim2),
        ]
    e_pad = n_tiles * tile
    cost = pl.CostEstimate(
        flops=int(2 * e_pad * two_f * (2 * n_nodes + we.shape[0] + feat)),
        transcendentals=int(e_pad * (we.shape[0] + 2 * feat)),
        bytes_accessed=int(e_pad * 12 + 4 * n_nodes * (two_f + 4 * feat)))
    return pl.pallas_call(
        functools.partial(body, tpc=tpc, n_nodes=n_nodes, feat=feat,
                          num_g=num_g, off_step=off_step, pad_off=pad_off,
                          coeff=coeff),
        out_shape=(jax.ShapeDtypeStruct((2, n_nodes, feat), jnp.float32),
                   jax.ShapeDtypeStruct((2, n_nodes, feat), jnp.float32)),
        grid=(2, tpc),
        in_specs=edge_specs + extra_specs + [
            pl.BlockSpec((feat, 2 * two_f), cim2),
            pl.BlockSpec((1, 2 * two_f), cim2),
            pl.BlockSpec(we.shape, cim2),
            pl.BlockSpec((1, two_f), cim2),
            pl.BlockSpec((1, two_f), cim2),
        ],
        out_specs=(pl.BlockSpec((1, n_nodes, feat), lambda c, t: (c, 0, 0)),
                   pl.BlockSpec((1, n_nodes, feat), lambda c, t: (c, 0, 0))),
        scratch_shapes=[pltpu.VMEM((n_nodes, 2 * two_f), jnp.float32),
                        pltpu.VMEM((n_nodes, feat), jnp.float32)],
        compiler_params=pltpu.CompilerParams(
            dimension_semantics=("parallel", "arbitrary"),
            vmem_limit_bytes=_VMEM_LIMIT),
        cost_estimate=cost,
    )(dstr, srcr, pt, *extra_in, wn.astype(jnp.bfloat16),
      nb.reshape(1, 2 * two_f), we.astype(jnp.bfloat16),
      sc.reshape(1, two_f), sh.reshape(1, two_f))


def _head_call(part_prev, x_prev, lg, lb, pool, hidden, w_out, b_out, *,
               n_nodes, feat):
    n_graphs = pool.shape[0]
    n_targets = w_out.shape[1]
    cim2 = lambda: (0, 0)
    inputs = [part_prev, x_prev, lg.reshape(1, feat), lb.reshape(1, feat),
              pool]
    in_specs = [
        pl.BlockSpec((2, n_nodes, feat), lambda: (0, 0, 0)),
        pl.BlockSpec((1, n_nodes, feat), lambda: (0, 0, 0)),
        pl.BlockSpec((1, feat), cim2),
        pl.BlockSpec((1, feat), cim2),
        pl.BlockSpec(pool.shape, cim2),
    ]
    for w, b in hidden:
        inputs += [w.astype(jnp.bfloat16), b.reshape(1, -1)]
        in_specs += [pl.BlockSpec(w.shape, cim2),
                     pl.BlockSpec((1, b.shape[0]), cim2)]
    inputs += [w_out.astype(jnp.bfloat16), b_out.reshape(1, n_targets)]
    in_specs += [pl.BlockSpec(w_out.shape, cim2),
                 pl.BlockSpec((1, n_targets), cim2)]
    return pl.pallas_call(
        functools.partial(_head_kernel, num_hidden=len(hidden)),
        out_shape=jax.ShapeDtypeStruct((n_graphs, n_targets), jnp.float32),
        in_specs=in_specs,
        out_specs=pl.BlockSpec((n_graphs, n_targets), cim2),
        compiler_params=pltpu.CompilerParams(vmem_limit_bytes=_VMEM_LIMIT),
    )(*inputs)


def kernel(embedding, nuc_w, nuc_b,
           conv0_w_node, conv0_node_b, conv0_w_edge, conv0_scale, conv0_shift, conv0_ln_gamma, conv0_ln_beta,
           conv1_w_node, conv1_node_b, conv1_w_edge, conv1_scale, conv1_shift, conv1_ln_gamma, conv1_ln_beta,
           conv2_w_node, conv2_node_b, conv2_w_edge, conv2_scale, conv2_shift, conv2_ln_gamma, conv2_ln_beta,
           conv3_w_node, conv3_node_b, conv3_w_edge, conv3_scale, conv3_shift, conv3_ln_gamma, conv3_ln_beta,
           conv4_w_node, conv4_node_b, conv4_w_edge, conv4_scale, conv4_shift, conv4_ln_gamma, conv4_ln_beta,
           conv5_w_node, conv5_node_b, conv5_w_edge, conv5_scale, conv5_shift, conv5_ln_gamma, conv5_ln_beta,
           conv_to_fc_w, conv_to_fc_b,
           fc0_w, fc0_b, fc1_w, fc1_b, fc2_w, fc2_b,
           out_w, out_b,
           atomic_numbers, positions, batch, neighbors_index):
    num_graphs = 2
    num_g = 100
    cutoff = 6.0
    n_nodes = atomic_numbers.shape[0]
    feat = nuc_w.shape[1]
    two_f = 2 * feat

    convs = [
        (conv0_w_node, conv0_node_b, conv0_w_edge, conv0_scale, conv0_shift, conv0_ln_gamma, conv0_ln_beta),
        (conv1_w_node, conv1_node_b, conv1_w_edge, conv1_scale, conv1_shift, conv1_ln_gamma, conv1_ln_beta),
        (conv2_w_node, conv2_node_b, conv2_w_edge, conv2_scale, conv2_shift, conv2_ln_gamma, conv2_ln_beta),
        (conv3_w_node, conv3_node_b, conv3_w_edge, conv3_scale, conv3_shift, conv3_ln_gamma, conv3_ln_beta),
        (conv4_w_node, conv4_node_b, conv4_w_edge, conv4_scale, conv4_shift, conv4_ln_gamma, conv4_ln_beta),
        (conv5_w_node, conv5_node_b, conv5_w_edge, conv5_scale, conv5_shift, conv5_ln_gamma, conv5_ln_beta),
    ]

    src = neighbors_index[0]
    dst = neighbors_index[1]
    n_edges = src.shape[0]

    sigma = cutoff / (num_g - 1)
    coeff = -0.5 / float(sigma ** 2)
    off_step = sigma
    pad_off = 1e3
    pt = jnp.zeros((8, n_nodes), jnp.float32).at[:3].set(
        positions.astype(jnp.float32).T)

    tile = _EDGE_TILE
    n_tiles = _round_up(_round_up(max(n_edges, 1), tile) // tile, 2)
    e_pad = n_tiles * tile
    if e_pad != n_edges:
        p = e_pad - n_edges
        src = jnp.concatenate([src, jnp.zeros((p,), src.dtype)])
        dst = jnp.concatenate([dst, jnp.full((p,), -1, dst.dtype)])
    tpc = n_tiles // 2
    dstr = dst.reshape(n_tiles, 1, tile)
    srcr = src.reshape(n_tiles, 1, tile)

    embx = embedding[atomic_numbers - 1]

    onehot = (batch[None, :] == jnp.arange(num_graphs,
                                           dtype=batch.dtype)[:, None]
              ).astype(jnp.float32)
    pool = onehot / jnp.maximum(onehot.sum(axis=1, keepdims=True), 1.0)

    prev = (None, embx, nuc_w, nuc_b)
    for k, (wn, nb, we, sc, sh, lg, lb) in enumerate(convs):
        part, xout = _conv_call(
            dstr, srcr, pt, prev, wn, nb, we, sc, sh, tpc=tpc, tile=tile,
            n_nodes=n_nodes, feat=feat, num_g=num_g, off_step=off_step,
            pad_off=pad_off, coeff=coeff)
        prev = (part, xout[:1], lg, lb)

    hidden = [(conv_to_fc_w, conv_to_fc_b), (fc0_w, fc0_b),
              (fc1_w, fc1_b), (fc2_w, fc2_b)]
    part, x_prev, lg, lb = prev
    return _head_call(part, x_prev, lg, lb, pool, hidden, out_w, out_b,
                      n_nodes=n_nodes, feat=feat)


# speedup vs baseline: 8.2862x; 1.2343x over previous
import functools

import jax
import jax.numpy as jnp
from jax import lax
from jax.experimental import pallas as pl
from jax.experimental.pallas import tpu as pltpu

_VMEM_LIMIT = 48 * 1024 * 1024
_EDGE_TILE = 4096


def _round_up(x, m):
    return ((x + m - 1) // m) * m


def _softplus(x):
    return jnp.maximum(x, 0.0) + jnp.log1p(jnp.exp(-jnp.abs(x)))


def _edge_step(dst_ref, src_ref, pt_ref, we_ref, sc_ref, sh_ref, h_scr,
               acc_ref, *, n_nodes, feat, num_g, off_step, pad_off, coeff):
    two_f = 2 * feat
    n2 = 2 * n_nodes
    row2 = lax.broadcasted_iota(jnp.int32, (n2, 1), 0)
    tgt = jnp.where(row2 < n_nodes, dst_ref[0], src_ref[0])
    oh2 = (row2 == tgt).astype(jnp.float32)
    dims = (((0,), (0,)), ((), ()))
    zfull = lax.dot_general(oh2, h_scr[...], dims,
                            preferred_element_type=jnp.float32,
                            precision=lax.Precision.HIGHEST)
    diff = zfull[:, two_f:]
    d = jnp.sqrt(jnp.sum(diff * diff, axis=1, keepdims=True))
    gcol = lax.broadcasted_iota(jnp.int32, (1, we_ref.shape[0]), 1
                                ).astype(jnp.float32)
    off = jnp.where(gcol < num_g, gcol * off_step, pad_off)
    u = d - off
    ef = jnp.exp(coeff * (u * u)).astype(jnp.bfloat16)
    z = (zfull[:, :two_f]
         + jnp.dot(ef, we_ref[...], preferred_element_type=jnp.float32))
    z = z * sc_ref[...] + sh_ref[...]
    gate = pl.reciprocal(1.0 + jnp.exp(-z[:, :feat]), approx=True)
    val = _softplus(z[:, feat:])
    msg = (gate * val).astype(jnp.bfloat16)
    acc_ref[...] += jnp.dot(oh2[:n_nodes].astype(jnp.bfloat16), msg,
                            preferred_element_type=jnp.float32)


def _finalize(part0, part1, x_prev, gamma, beta):
    a = part0 + part1
    mean = jnp.mean(a, axis=-1, keepdims=True)
    var = jnp.mean((a - mean) ** 2, axis=-1, keepdims=True)
    ln = (a - mean) * lax.rsqrt(var + 1e-5) * gamma + beta
    return _softplus(ln + x_prev)


def _conv0_kernel(dst_ref, src_ref, pt_ref, embx_ref, nucw_ref, nucb_ref,
                  wn_ref, nb_ref, we_ref, sc_ref, sh_ref, part_ref, xout_ref,
                  h_scr, acc_ref, *, tpc, n_nodes, feat, num_g, off_step,
                  pad_off, coeff):
    t = pl.program_id(1)

    @pl.when(t == 0)
    def _():
        x = jnp.dot(embx_ref[...], nucw_ref[...],
                    preferred_element_type=jnp.float32) + nucb_ref[...]
        xout_ref[0] = x
        h = jnp.dot(x.astype(jnp.bfloat16), wn_ref[...],
                    preferred_element_type=jnp.float32) + nb_ref[...]
        two_f = 2 * feat
        h_scr[:n_nodes, :two_f] = h[:, :two_f]
        h_scr[n_nodes:, :two_f] = h[:, two_f:]
        h_scr[:n_nodes, two_f:] = pt_ref[...]
        h_scr[n_nodes:, two_f:] = -pt_ref[...]
        acc_ref[...] = jnp.zeros_like(acc_ref)

    _edge_step(dst_ref, src_ref, pt_ref, we_ref, sc_ref, sh_ref, h_scr,
               acc_ref, n_nodes=n_nodes, feat=feat, num_g=num_g,
               off_step=off_step, pad_off=pad_off, coeff=coeff)

    @pl.when(t == tpc - 1)
    def _():
        part_ref[0] = acc_ref[...]


def _conv_kernel(dst_ref, src_ref, pt_ref, pprev_ref, xprev_ref, g_ref, b_ref,
                 wn_ref, nb_ref, we_ref, sc_ref, sh_ref, part_ref, xout_ref,
                 h_scr, acc_ref, *, tpc, n_nodes, feat, num_g, off_step,
                 pad_off, coeff):
    t = pl.program_id(1)

    @pl.when(t == 0)
    def _():
        x = _finalize(pprev_ref[0], pprev_ref[1], xprev_ref[0],
                      g_ref[...], b_ref[...])
        xout_ref[0] = x
        h = jnp.dot(x.astype(jnp.bfloat16), wn_ref[...],
                    preferred_element_type=jnp.float32) + nb_ref[...]
        two_f = 2 * feat
        h_scr[:n_nodes, :two_f] = h[:, :two_f]
        h_scr[n_nodes:, :two_f] = h[:, two_f:]
        h_scr[:n_nodes, two_f:] = pt_ref[...]
        h_scr[n_nodes:, two_f:] = -pt_ref[...]
        acc_ref[...] = jnp.zeros_like(acc_ref)

    _edge_step(dst_ref, src_ref, pt_ref, we_ref, sc_ref, sh_ref, h_scr,
               acc_ref, n_nodes=n_nodes, feat=feat, num_g=num_g,
               off_step=off_step, pad_off=pad_off, coeff=coeff)

    @pl.when(t == tpc - 1)
    def _():
        part_ref[0] = acc_ref[...]


def _head_kernel(*refs, num_hidden):
    pprev_ref, xprev_ref, g_ref, b_ref, pool_ref = refs[:5]
    x = _finalize(pprev_ref[0], pprev_ref[1], xprev_ref[0],
                  g_ref[...], b_ref[...])
    h = jnp.dot(pool_ref[...], x, preferred_element_type=jnp.float32)
    idx = 5
    for _ in range(num_hidden):
        w = refs[idx][...]
        b = refs[idx + 1][...]
        idx += 2
        h = _softplus(jnp.dot(h.astype(jnp.bfloat16), w,
                              preferred_element_type=jnp.float32) + b)
    w = refs[idx][...]
    b = refs[idx + 1][...]
    o_ref = refs[idx + 2]
    o_ref[...] = jnp.dot(h.astype(jnp.bfloat16), w,
                         preferred_element_type=jnp.float32) + b


def _conv_call(dstr, srcr, pt, prev, wn, nb, we, sc, sh, *, tpc, tile,
               n_nodes, feat, num_g, off_step, pad_off, coeff):
    two_f = 2 * feat
    n_tiles = dstr.shape[0]
    eim = lambda c, t: (c * tpc + t, 0, 0)
    cim2 = lambda c, t: (0, 0)
    cim3 = lambda c, t: (0, 0, 0)
    edge_specs = [pl.BlockSpec((1, 1, tile), eim) for _ in range(2)]
    edge_specs.append(pl.BlockSpec(pt.shape, cim2))
    if prev[0] is None:
        embx, nucw, nucb = prev[1], prev[2], prev[3]
        body = _conv0_kernel
        extra_in = [embx.astype(jnp.bfloat16), nucw.astype(jnp.bfloat16),
                    nucb.reshape(1, feat)]
        extra_specs = [
            pl.BlockSpec(embx.shape, cim2),
            pl.BlockSpec(nucw.shape, cim2),
            pl.BlockSpec((1, feat), cim2),
        ]
    else:
        part_prev, x_prev, lg, lb = prev
        body = _conv_kernel
        extra_in = [part_prev, x_prev, lg.reshape(1, feat), lb.reshape(1, feat)]
        extra_specs = [
            pl.BlockSpec((2, n_nodes, feat), cim3),
            pl.BlockSpec((1, n_nodes, feat), cim3),
            pl.BlockSpec((1, feat), cim2),
            pl.BlockSpec((1, feat), cim2),
        ]
    e_pad = n_tiles * tile
    cost = pl.CostEstimate(
        flops=int(2 * e_pad * two_f * (2 * n_nodes + we.shape[0] + feat)),
        transcendentals=int(e_pad * (we.shape[0] + 2 * feat)),
        bytes_accessed=int(e_pad * 12 + 4 * n_nodes * (two_f + 4 * feat)))
    return pl.pallas_call(
        functools.partial(body, tpc=tpc, n_nodes=n_nodes, feat=feat,
                          num_g=num_g, off_step=off_step, pad_off=pad_off,
                          coeff=coeff),
        out_shape=(jax.ShapeDtypeStruct((2, n_nodes, feat), jnp.float32),
                   jax.ShapeDtypeStruct((2, n_nodes, feat), jnp.float32)),
        grid=(2, tpc),
        in_specs=edge_specs + extra_specs + [
            pl.BlockSpec((feat, 2 * two_f), cim2),
            pl.BlockSpec((1, 2 * two_f), cim2),
            pl.BlockSpec(we.shape, cim2),
            pl.BlockSpec((1, two_f), cim2),
            pl.BlockSpec((1, two_f), cim2),
        ],
        out_specs=(pl.BlockSpec((1, n_nodes, feat), lambda c, t: (c, 0, 0)),
                   pl.BlockSpec((1, n_nodes, feat), lambda c, t: (c, 0, 0))),
        scratch_shapes=[pltpu.VMEM((2 * n_nodes, two_f + 8), jnp.float32),
                        pltpu.VMEM((n_nodes, feat), jnp.float32)],
        compiler_params=pltpu.CompilerParams(
            dimension_semantics=("parallel", "arbitrary"),
            vmem_limit_bytes=_VMEM_LIMIT),
        cost_estimate=cost,
    )(dstr, srcr, pt, *extra_in, wn.astype(jnp.bfloat16),
      nb.reshape(1, 2 * two_f), we.astype(jnp.bfloat16),
      sc.reshape(1, two_f), sh.reshape(1, two_f))


def _head_call(part_prev, x_prev, lg, lb, pool, hidden, w_out, b_out, *,
               n_nodes, feat):
    n_graphs = pool.shape[0]
    n_targets = w_out.shape[1]
    cim2 = lambda: (0, 0)
    inputs = [part_prev, x_prev, lg.reshape(1, feat), lb.reshape(1, feat),
              pool]
    in_specs = [
        pl.BlockSpec((2, n_nodes, feat), lambda: (0, 0, 0)),
        pl.BlockSpec((1, n_nodes, feat), lambda: (0, 0, 0)),
        pl.BlockSpec((1, feat), cim2),
        pl.BlockSpec((1, feat), cim2),
        pl.BlockSpec(pool.shape, cim2),
    ]
    for w, b in hidden:
        inputs += [w.astype(jnp.bfloat16), b.reshape(1, -1)]
        in_specs += [pl.BlockSpec(w.shape, cim2),
                     pl.BlockSpec((1, b.shape[0]), cim2)]
    inputs += [w_out.astype(jnp.bfloat16), b_out.reshape(1, n_targets)]
    in_specs += [pl.BlockSpec(w_out.shape, cim2),
                 pl.BlockSpec((1, n_targets), cim2)]
    return pl.pallas_call(
        functools.partial(_head_kernel, num_hidden=len(hidden)),
        out_shape=jax.ShapeDtypeStruct((n_graphs, n_targets), jnp.float32),
        in_specs=in_specs,
        out_specs=pl.BlockSpec((n_graphs, n_targets), cim2),
        compiler_params=pltpu.CompilerParams(vmem_limit_bytes=_VMEM_LIMIT),
    )(*inputs)


def kernel(embedding, nuc_w, nuc_b,
           conv0_w_node, conv0_node_b, conv0_w_edge, conv0_scale, conv0_shift, conv0_ln_gamma, conv0_ln_beta,
           conv1_w_node, conv1_node_b, conv1_w_edge, conv1_scale, conv1_shift, conv1_ln_gamma, conv1_ln_beta,
           conv2_w_node, conv2_node_b, conv2_w_edge, conv2_scale, conv2_shift, conv2_ln_gamma, conv2_ln_beta,
           conv3_w_node, conv3_node_b, conv3_w_edge, conv3_scale, conv3_shift, conv3_ln_gamma, conv3_ln_beta,
           conv4_w_node, conv4_node_b, conv4_w_edge, conv4_scale, conv4_shift, conv4_ln_gamma, conv4_ln_beta,
           conv5_w_node, conv5_node_b, conv5_w_edge, conv5_scale, conv5_shift, conv5_ln_gamma, conv5_ln_beta,
           conv_to_fc_w, conv_to_fc_b,
           fc0_w, fc0_b, fc1_w, fc1_b, fc2_w, fc2_b,
           out_w, out_b,
           atomic_numbers, positions, batch, neighbors_index):
    num_graphs = 2
    num_g = 100
    cutoff = 6.0
    n_nodes = atomic_numbers.shape[0]
    feat = nuc_w.shape[1]
    two_f = 2 * feat

    convs = [
        (conv0_w_node, conv0_node_b, conv0_w_edge, conv0_scale, conv0_shift, conv0_ln_gamma, conv0_ln_beta),
        (conv1_w_node, conv1_node_b, conv1_w_edge, conv1_scale, conv1_shift, conv1_ln_gamma, conv1_ln_beta),
        (conv2_w_node, conv2_node_b, conv2_w_edge, conv2_scale, conv2_shift, conv2_ln_gamma, conv2_ln_beta),
        (conv3_w_node, conv3_node_b, conv3_w_edge, conv3_scale, conv3_shift, conv3_ln_gamma, conv3_ln_beta),
        (conv4_w_node, conv4_node_b, conv4_w_edge, conv4_scale, conv4_shift, conv4_ln_gamma, conv4_ln_beta),
        (conv5_w_node, conv5_node_b, conv5_w_edge, conv5_scale, conv5_shift, conv5_ln_gamma, conv5_ln_beta),
    ]

    src = neighbors_index[0]
    dst = neighbors_index[1]
    n_edges = src.shape[0]

    sigma = cutoff / (num_g - 1)
    coeff = -0.5 / float(sigma ** 2)
    off_step = sigma
    pad_off = 1e3
    pt = jnp.zeros((n_nodes, 8), jnp.float32).at[:, :3].set(
        positions.astype(jnp.float32))

    tile = _EDGE_TILE
    n_tiles = _round_up(_round_up(max(n_edges, 1), tile) // tile, 2)
    e_pad = n_tiles * tile
    if e_pad != n_edges:
        p = e_pad - n_edges
        src = jnp.concatenate([src, jnp.zeros((p,), src.dtype)])
        dst = jnp.concatenate([dst, jnp.full((p,), -1, dst.dtype)])
    tpc = n_tiles // 2
    dstr = dst.reshape(n_tiles, 1, tile)
    srcr = (src + n_nodes).reshape(n_tiles, 1, tile)

    embx = embedding[atomic_numbers - 1]

    onehot = (batch[None, :] == jnp.arange(num_graphs,
                                           dtype=batch.dtype)[:, None]
              ).astype(jnp.float32)
    pool = onehot / jnp.maximum(onehot.sum(axis=1, keepdims=True), 1.0)

    prev = (None, embx, nuc_w, nuc_b)
    for k, (wn, nb, we, sc, sh, lg, lb) in enumerate(convs):
        part, xout = _conv_call(
            dstr, srcr, pt, prev, wn, nb, we, sc, sh, tpc=tpc, tile=tile,
            n_nodes=n_nodes, feat=feat, num_g=num_g, off_step=off_step,
            pad_off=pad_off, coeff=coeff)
        prev = (part, xout[:1], lg, lb)

    hidden = [(conv_to_fc_w, conv_to_fc_b), (fc0_w, fc0_b),
              (fc1_w, fc1_b), (fc2_w, fc2_b)]
    part, x_prev, lg, lb = prev
    return _head_call(part, x_prev, lg, lb, pool, hidden, out_w, out_b,
                      n_nodes=n_nodes, feat=feat)


# speedup vs baseline: 10.8916x; 1.3144x over previous
import functools

import jax
import jax.numpy as jnp
from jax import lax
from jax.experimental import pallas as pl
from jax.experimental.pallas import tpu as pltpu

_VMEM_LIMIT = 48 * 1024 * 1024
_EDGE_TILE = 4096


def _round_up(x, m):
    return ((x + m - 1) // m) * m


def _softplus(x):
    return jnp.maximum(x, 0.0) + jnp.log1p(jnp.exp(-jnp.abs(x)))


def _edge_step(dst_ref, src_ref, we_ref, sc_ref, sh_ref, hhi_scr,
               hmid_scr, hlo_scr, acc_ref, *, n_nodes, feat, num_g, off_step,
               pad_off, coeff):
    two_f = 2 * feat
    n2 = 2 * n_nodes
    row2 = lax.broadcasted_iota(jnp.int32, (n2, 1), 0)
    tgt = jnp.where(row2 < n_nodes, dst_ref[0], src_ref[0])
    oh2 = (row2 == tgt).astype(jnp.bfloat16)
    dims = (((0,), (0,)), ((), ()))
    zfull = (lax.dot_general(oh2, hhi_scr[...], dims,
                             preferred_element_type=jnp.float32)
             + lax.dot_general(oh2, hmid_scr[...], dims,
                               preferred_element_type=jnp.float32)
             + lax.dot_general(oh2, hlo_scr[...], dims,
                               preferred_element_type=jnp.float32))
    diff = zfull[:, two_f:]
    d = jnp.sqrt(diff[:, 0:1] * diff[:, 0:1]
                 + diff[:, 1:2] * diff[:, 1:2]
                 + diff[:, 2:3] * diff[:, 2:3])
    gcol = lax.broadcasted_iota(jnp.int32, (1, we_ref.shape[0]), 1
                                ).astype(jnp.float32)
    off = jnp.where(gcol < num_g, gcol * off_step, pad_off)
    u = d - off
    ef = jnp.exp(coeff * (u * u)).astype(jnp.bfloat16)
    z = (zfull[:, :two_f]
         + jnp.dot(ef, we_ref[...], preferred_element_type=jnp.float32))
    z = z * sc_ref[...] + sh_ref[...]
    gate = pl.reciprocal(1.0 + jnp.exp(-z[:, :feat]), approx=True)
    val = _softplus(z[:, feat:])
    msg = (gate * val).astype(jnp.bfloat16)
    acc_ref[...] += jnp.dot(oh2[:n_nodes], msg,
                            preferred_element_type=jnp.float32)


def _finalize(part0, part1, x_prev, gamma, beta):
    a = part0 + part1
    mean = jnp.mean(a, axis=-1, keepdims=True)
    var = jnp.mean((a - mean) ** 2, axis=-1, keepdims=True)
    ln = (a - mean) * lax.rsqrt(var + 1e-5) * gamma + beta
    return _softplus(ln + x_prev)


def _conv0_kernel(dst_ref, src_ref, pt_ref, embx_ref, nucw_ref, nucb_ref,
                  wn_ref, nb_ref, we_ref, sc_ref, sh_ref, part_ref, xout_ref,
                  hhi_scr, hmid_scr, hlo_scr, acc_ref, *, tpc, n_nodes,
                  feat, num_g, off_step, pad_off, coeff):
    t = pl.program_id(1)

    @pl.when(t == 0)
    def _():
        x = jnp.dot(embx_ref[...], nucw_ref[...],
                    preferred_element_type=jnp.float32) + nucb_ref[...]
        xout_ref[0] = x
        h = jnp.dot(x.astype(jnp.bfloat16), wn_ref[...],
                    preferred_element_type=jnp.float32) + nb_ref[...]
        two_f = 2 * feat
        pt = pt_ref[...]
        tab = jnp.concatenate(
            [jnp.concatenate([h[:, :two_f], pt], axis=1),
             jnp.concatenate([h[:, two_f:], -pt], axis=1)], axis=0)
        thi = tab.astype(jnp.bfloat16)
        r1 = tab - thi.astype(jnp.float32)
        tmid = r1.astype(jnp.bfloat16)
        hhi_scr[...] = thi
        hmid_scr[...] = tmid
        hlo_scr[...] = (r1 - tmid.astype(jnp.float32)).astype(jnp.bfloat16)
        acc_ref[...] = jnp.zeros_like(acc_ref)

    _edge_step(dst_ref, src_ref, we_ref, sc_ref, sh_ref, hhi_scr,
               hmid_scr, hlo_scr, acc_ref, n_nodes=n_nodes, feat=feat,
               num_g=num_g, off_step=off_step, pad_off=pad_off, coeff=coeff)

    @pl.when(t == tpc - 1)
    def _():
        part_ref[0] = acc_ref[...]


def _conv_kernel(dst_ref, src_ref, pt_ref, pprev_ref, xprev_ref, g_ref, b_ref,
                 wn_ref, nb_ref, we_ref, sc_ref, sh_ref, part_ref, xout_ref,
                 hhi_scr, hmid_scr, hlo_scr, acc_ref, *, tpc, n_nodes,
                 feat, num_g, off_step, pad_off, coeff):
    t = pl.program_id(1)

    @pl.when(t == 0)
    def _():
        x = _finalize(pprev_ref[0], pprev_ref[1], xprev_ref[0],
                      g_ref[...], b_ref[...])
        xout_ref[0] = x
        h = jnp.dot(x.astype(jnp.bfloat16), wn_ref[...],
                    preferred_element_type=jnp.float32) + nb_ref[...]
        two_f = 2 * feat
        pt = pt_ref[...]
        tab = jnp.concatenate(
            [jnp.concatenate([h[:, :two_f], pt], axis=1),
             jnp.concatenate([h[:, two_f:], -pt], axis=1)], axis=0)
        thi = tab.astype(jnp.bfloat16)
        r1 = tab - thi.astype(jnp.float32)
        tmid = r1.astype(jnp.bfloat16)
        hhi_scr[...] = thi
        hmid_scr[...] = tmid
        hlo_scr[...] = (r1 - tmid.astype(jnp.float32)).astype(jnp.bfloat16)
        acc_ref[...] = jnp.zeros_like(acc_ref)

    _edge_step(dst_ref, src_ref, we_ref, sc_ref, sh_ref, hhi_scr,
               hmid_scr, hlo_scr, acc_ref, n_nodes=n_nodes, feat=feat,
               num_g=num_g, off_step=off_step, pad_off=pad_off, coeff=coeff)

    @pl.when(t == tpc - 1)
    def _():
        part_ref[0] = acc_ref[...]


def _head_kernel(*refs, num_hidden):
    pprev_ref, xprev_ref, g_ref, b_ref, pool_ref = refs[:5]
    x = _finalize(pprev_ref[0], pprev_ref[1], xprev_ref[0],
                  g_ref[...], b_ref[...])
    h = jnp.dot(pool_ref[...], x, preferred_element_type=jnp.float32)
    idx = 5
    for _ in range(num_hidden):
        w = refs[idx][...]
        b = refs[idx + 1][...]
        idx += 2
        h = _softplus(jnp.dot(h.astype(jnp.bfloat16), w,
                              preferred_element_type=jnp.float32) + b)
    w = refs[idx][...]
    b = refs[idx + 1][...]
    o_ref = refs[idx + 2]
    o_ref[...] = jnp.dot(h.astype(jnp.bfloat16), w,
                         preferred_element_type=jnp.float32) + b


def _conv_call(dstr, srcr, pt, prev, wn, nb, we, sc, sh, *, tpc, tile,
               n_nodes, feat, num_g, off_step, pad_off, coeff):
    two_f = 2 * feat
    n_tiles = dstr.shape[0]
    eim = lambda c, t: (c * tpc + t, 0, 0)
    cim2 = lambda c, t: (0, 0)
    cim3 = lambda c, t: (0, 0, 0)
    edge_specs = [pl.BlockSpec((1, 1, tile), eim) for _ in range(2)]
    edge_specs.append(pl.BlockSpec(pt.shape, cim2))
    if prev[0] is None:
        embx, nucw, nucb = prev[1], prev[2], prev[3]
        body = _conv0_kernel
        extra_in = [embx.astype(jnp.bfloat16), nucw.astype(jnp.bfloat16),
                    nucb.reshape(1, feat)]
        extra_specs = [
            pl.BlockSpec(embx.shape, cim2),
            pl.BlockSpec(nucw.shape, cim2),
            pl.BlockSpec((1, feat), cim2),
        ]
    else:
        part_prev, x_prev, lg, lb = prev
        body = _conv_kernel
        extra_in = [part_prev, x_prev, lg.reshape(1, feat), lb.reshape(1, feat)]
        extra_specs = [
            pl.BlockSpec((2, n_nodes, feat), cim3),
            pl.BlockSpec((1, n_nodes, feat), cim3),
            pl.BlockSpec((1, feat), cim2),
            pl.BlockSpec((1, feat), cim2),
        ]
    e_pad = n_tiles * tile
    cost = pl.CostEstimate(
        flops=int(2 * e_pad * two_f * (2 * n_nodes + we.shape[0] + feat)),
        transcendentals=int(e_pad * (we.shape[0] + 2 * feat)),
        bytes_accessed=int(e_pad * 12 + 4 * n_nodes * (two_f + 4 * feat)))
    return pl.pallas_call(
        functools.partial(body, tpc=tpc, n_nodes=n_nodes, feat=feat,
                          num_g=num_g, off_step=off_step, pad_off=pad_off,
                          coeff=coeff),
        out_shape=(jax.ShapeDtypeStruct((2, n_nodes, feat), jnp.float32),
                   jax.ShapeDtypeStruct((2, n_nodes, feat), jnp.float32)),
        grid=(2, tpc),
        in_specs=edge_specs + extra_specs + [
            pl.BlockSpec((feat, 2 * two_f), cim2),
            pl.BlockSpec((1, 2 * two_f), cim2),
            pl.BlockSpec(we.shape, cim2),
            pl.BlockSpec((1, two_f), cim2),
            pl.BlockSpec((1, two_f), cim2),
        ],
        out_specs=(pl.BlockSpec((1, n_nodes, feat), lambda c, t: (c, 0, 0)),
                   pl.BlockSpec((1, n_nodes, feat), lambda c, t: (c, 0, 0))),
        scratch_shapes=[pltpu.VMEM((2 * n_nodes, two_f + 8), jnp.bfloat16),
                        pltpu.VMEM((2 * n_nodes, two_f + 8), jnp.bfloat16),
                        pltpu.VMEM((2 * n_nodes, two_f + 8), jnp.bfloat16),
                        pltpu.VMEM((n_nodes, feat), jnp.float32)],
        compiler_params=pltpu.CompilerParams(
            dimension_semantics=("parallel", "arbitrary"),
            vmem_limit_bytes=_VMEM_LIMIT),
        cost_estimate=cost,
    )(dstr, srcr, pt, *extra_in, wn.astype(jnp.bfloat16),
      nb.reshape(1, 2 * two_f), we.astype(jnp.bfloat16),
      sc.reshape(1, two_f), sh.reshape(1, two_f))


def _head_call(part_prev, x_prev, lg, lb, pool, hidden, w_out, b_out, *,
               n_nodes, feat):
    n_graphs = pool.shape[0]
    n_targets = w_out.shape[1]
    cim2 = lambda: (0, 0)
    inputs = [part_prev, x_prev, lg.reshape(1, feat), lb.reshape(1, feat),
              pool]
    in_specs = [
        pl.BlockSpec((2, n_nodes, feat), lambda: (0, 0, 0)),
        pl.BlockSpec((1, n_nodes, feat), lambda: (0, 0, 0)),
        pl.BlockSpec((1, feat), cim2),
        pl.BlockSpec((1, feat), cim2),
        pl.BlockSpec(pool.shape, cim2),
    ]
    for w, b in hidden:
        inputs += [w.astype(jnp.bfloat16), b.reshape(1, -1)]
        in_specs += [pl.BlockSpec(w.shape, cim2),
                     pl.BlockSpec((1, b.shape[0]), cim2)]
    inputs += [w_out.astype(jnp.bfloat16), b_out.reshape(1, n_targets)]
    in_specs += [pl.BlockSpec(w_out.shape, cim2),
                 pl.BlockSpec((1, n_targets), cim2)]
    return pl.pallas_call(
        functools.partial(_head_kernel, num_hidden=len(hidden)),
        out_shape=jax.ShapeDtypeStruct((n_graphs, n_targets), jnp.float32),
        in_specs=in_specs,
        out_specs=pl.BlockSpec((n_graphs, n_targets), cim2),
        compiler_params=pltpu.CompilerParams(vmem_limit_bytes=_VMEM_LIMIT),
    )(*inputs)


def kernel(embedding, nuc_w, nuc_b,
           conv0_w_node, conv0_node_b, conv0_w_edge, conv0_scale, conv0_shift, conv0_ln_gamma, conv0_ln_beta,
           conv1_w_node, conv1_node_b, conv1_w_edge, conv1_scale, conv1_shift, conv1_ln_gamma, conv1_ln_beta,
           conv2_w_node, conv2_node_b, conv2_w_edge, conv2_scale, conv2_shift, conv2_ln_gamma, conv2_ln_beta,
           conv3_w_node, conv3_node_b, conv3_w_edge, conv3_scale, conv3_shift, conv3_ln_gamma, conv3_ln_beta,
           conv4_w_node, conv4_node_b, conv4_w_edge, conv4_scale, conv4_shift, conv4_ln_gamma, conv4_ln_beta,
           conv5_w_node, conv5_node_b, conv5_w_edge, conv5_scale, conv5_shift, conv5_ln_gamma, conv5_ln_beta,
           conv_to_fc_w, conv_to_fc_b,
           fc0_w, fc0_b, fc1_w, fc1_b, fc2_w, fc2_b,
           out_w, out_b,
           atomic_numbers, positions, batch, neighbors_index):
    num_graphs = 2
    num_g = 100
    cutoff = 6.0
    n_nodes = atomic_numbers.shape[0]
    feat = nuc_w.shape[1]
    two_f = 2 * feat

    convs = [
        (conv0_w_node, conv0_node_b, conv0_w_edge, conv0_scale, conv0_shift, conv0_ln_gamma, conv0_ln_beta),
        (conv1_w_node, conv1_node_b, conv1_w_edge, conv1_scale, conv1_shift, conv1_ln_gamma, conv1_ln_beta),
        (conv2_w_node, conv2_node_b, conv2_w_edge, conv2_scale, conv2_shift, conv2_ln_gamma, conv2_ln_beta),
        (conv3_w_node, conv3_node_b, conv3_w_edge, conv3_scale, conv3_shift, conv3_ln_gamma, conv3_ln_beta),
        (conv4_w_node, conv4_node_b, conv4_w_edge, conv4_scale, conv4_shift, conv4_ln_gamma, conv4_ln_beta),
        (conv5_w_node, conv5_node_b, conv5_w_edge, conv5_scale, conv5_shift, conv5_ln_gamma, conv5_ln_beta),
    ]

    src = neighbors_index[0]
    dst = neighbors_index[1]
    n_edges = src.shape[0]

    sigma = cutoff / (num_g - 1)
    coeff = -0.5 / float(sigma ** 2)
    off_step = sigma
    pad_off = 1e3
    pt = jnp.zeros((n_nodes, 8), jnp.float32).at[:, :3].set(
        positions.astype(jnp.float32))

    tile = _EDGE_TILE
    n_tiles = _round_up(_round_up(max(n_edges, 1), tile) // tile, 2)
    e_pad = n_tiles * tile
    if e_pad != n_edges:
        p = e_pad - n_edges
        src = jnp.concatenate([src, jnp.zeros((p,), src.dtype)])
        dst = jnp.concatenate([dst, jnp.full((p,), -1, dst.dtype)])
    tpc = n_tiles // 2
    dstr = dst.reshape(n_tiles, 1, tile)
    srcr = (src + n_nodes).reshape(n_tiles, 1, tile)

    embx = embedding[atomic_numbers - 1]

    onehot = (batch[None, :] == jnp.arange(num_graphs,
                                           dtype=batch.dtype)[:, None]
              ).astype(jnp.float32)
    pool = onehot / jnp.maximum(onehot.sum(axis=1, keepdims=True), 1.0)

    prev = (None, embx, nuc_w, nuc_b)
    for k, (wn, nb, we, sc, sh, lg, lb) in enumerate(convs):
        part, xout = _conv_call(
            dstr, srcr, pt, prev, wn, nb, we, sc, sh, tpc=tpc, tile=tile,
            n_nodes=n_nodes, feat=feat, num_g=num_g, off_step=off_step,
            pad_off=pad_off, coeff=coeff)
        prev = (part, xout[:1], lg, lb)

    hidden = [(conv_to_fc_w, conv_to_fc_b), (fc0_w, fc0_b),
              (fc1_w, fc1_b), (fc2_w, fc2_b)]
    part, x_prev, lg, lb = prev
    return _head_call(part, x_prev, lg, lb, pool, hidden, out_w, out_b,
                      n_nodes=n_nodes, feat=feat)


# speedup vs baseline: 12.3306x; 1.1321x over previous
import functools

import jax
import jax.numpy as jnp
from jax import lax
from jax.experimental import pallas as pl
from jax.experimental.pallas import tpu as pltpu

_VMEM_LIMIT = 48 * 1024 * 1024
_EDGE_TILE = 8192


def _round_up(x, m):
    return ((x + m - 1) // m) * m


def _softplus(x):
    return jnp.maximum(x, 0.0) + jnp.log1p(jnp.exp(-jnp.abs(x)))


def _edge_step(dst_ref, src_ref, we_ref, sc_ref, sh_ref, hhi_scr,
               hmid_scr, hlo_scr, acc_ref, *, n_nodes, feat, num_g, off_step,
               pad_off, coeff):
    two_f = 2 * feat
    n2 = 2 * n_nodes
    row2 = lax.broadcasted_iota(jnp.int32, (n2, 1), 0)
    tgt = jnp.where(row2 < n_nodes, dst_ref[0], src_ref[0])
    oh2 = (row2 == tgt).astype(jnp.bfloat16)
    dims = (((0,), (0,)), ((), ()))
    zfull = (lax.dot_general(oh2, hhi_scr[...], dims,
                             preferred_element_type=jnp.float32)
             + lax.dot_general(oh2, hmid_scr[...], dims,
                               preferred_element_type=jnp.float32)
             + lax.dot_general(oh2, hlo_scr[...], dims,
                               preferred_element_type=jnp.float32))
    diff = zfull[:, two_f:]
    sq = diff * diff
    d = jnp.sqrt(jnp.sum(sq, axis=1, keepdims=True))
    gcol = lax.broadcasted_iota(jnp.int32, (1, we_ref.shape[0]), 1
                                ).astype(jnp.float32)
    off = jnp.where(gcol < num_g, gcol * off_step, pad_off)
    u = d - off
    ef = jnp.exp(coeff * (u * u)).astype(jnp.bfloat16)
    z = (zfull[:, :two_f]
         + jnp.dot(ef, we_ref[...], preferred_element_type=jnp.float32))
    z = z * sc_ref[...] + sh_ref[...]
    gate = pl.reciprocal(1.0 + jnp.exp(-z[:, :feat]), approx=True)
    val = _softplus(z[:, feat:])
    msg = (gate * val).astype(jnp.bfloat16)
    acc_ref[...] += jnp.dot(oh2[:n_nodes], msg,
                            preferred_element_type=jnp.float32)


def _finalize(part0, part1, x_prev, gamma, beta):
    a = part0 + part1
    mean = jnp.mean(a, axis=-1, keepdims=True)
    var = jnp.mean((a - mean) ** 2, axis=-1, keepdims=True)
    ln = (a - mean) * lax.rsqrt(var + 1e-5) * gamma + beta
    return _softplus(ln + x_prev)


def _conv0_kernel(dst_ref, src_ref, pt_ref, embx_ref, nucw_ref, nucb_ref,
                  wn_ref, nb_ref, we_ref, sc_ref, sh_ref, part_ref, xout_ref,
                  hhi_scr, hmid_scr, hlo_scr, acc_ref, *, tpc, n_nodes,
                  feat, num_g, off_step, pad_off, coeff):
    t = pl.program_id(1)

    @pl.when(t == 0)
    def _():
        x = jnp.dot(embx_ref[...], nucw_ref[...],
                    preferred_element_type=jnp.float32) + nucb_ref[...]
        xout_ref[0] = x
        h = jnp.dot(x.astype(jnp.bfloat16), wn_ref[...],
                    preferred_element_type=jnp.float32) + nb_ref[...]
        two_f = 2 * feat
        pt = pt_ref[...]
        tab = jnp.concatenate(
            [jnp.concatenate([h[:, :two_f], pt], axis=1),
             jnp.concatenate([h[:, two_f:], -pt], axis=1)], axis=0)
        thi = tab.astype(jnp.bfloat16)
        r1 = tab - thi.astype(jnp.float32)
        tmid = r1.astype(jnp.bfloat16)
        hhi_scr[...] = thi
        hmid_scr[...] = tmid
        hlo_scr[...] = (r1 - tmid.astype(jnp.float32)).astype(jnp.bfloat16)
        acc_ref[...] = jnp.zeros_like(acc_ref)

    _edge_step(dst_ref, src_ref, we_ref, sc_ref, sh_ref, hhi_scr,
               hmid_scr, hlo_scr, acc_ref, n_nodes=n_nodes, feat=feat,
               num_g=num_g, off_step=off_step, pad_off=pad_off, coeff=coeff)

    @pl.when(t == tpc - 1)
    def _():
        part_ref[0] = acc_ref[...]


def _conv_kernel(dst_ref, src_ref, pt_ref, pprev_ref, xprev_ref, g_ref, b_ref,
                 wn_ref, nb_ref, we_ref, sc_ref, sh_ref, part_ref, xout_ref,
                 hhi_scr, hmid_scr, hlo_scr, acc_ref, *, tpc, n_nodes,
                 feat, num_g, off_step, pad_off, coeff):
    t = pl.program_id(1)

    @pl.when(t == 0)
    def _():
        x = _finalize(pprev_ref[0], pprev_ref[1], xprev_ref[0],
                      g_ref[...], b_ref[...])
        xout_ref[0] = x
        h = jnp.dot(x.astype(jnp.bfloat16), wn_ref[...],
                    preferred_element_type=jnp.float32) + nb_ref[...]
        two_f = 2 * feat
        pt = pt_ref[...]
        tab = jnp.concatenate(
            [jnp.concatenate([h[:, :two_f], pt], axis=1),
             jnp.concatenate([h[:, two_f:], -pt], axis=1)], axis=0)
        thi = tab.astype(jnp.bfloat16)
        r1 = tab - thi.astype(jnp.float32)
        tmid = r1.astype(jnp.bfloat16)
        hhi_scr[...] = thi
        hmid_scr[...] = tmid
        hlo_scr[...] = (r1 - tmid.astype(jnp.float32)).astype(jnp.bfloat16)
        acc_ref[...] = jnp.zeros_like(acc_ref)

    _edge_step(dst_ref, src_ref, we_ref, sc_ref, sh_ref, hhi_scr,
               hmid_scr, hlo_scr, acc_ref, n_nodes=n_nodes, feat=feat,
               num_g=num_g, off_step=off_step, pad_off=pad_off, coeff=coeff)

    @pl.when(t == tpc - 1)
    def _():
        part_ref[0] = acc_ref[...]


def _head_kernel(*refs, num_hidden):
    pprev_ref, xprev_ref, g_ref, b_ref, pool_ref = refs[:5]
    x = _finalize(pprev_ref[0], pprev_ref[1], xprev_ref[0],
                  g_ref[...], b_ref[...])
    h = jnp.dot(pool_ref[...], x, preferred_element_type=jnp.float32)
    idx = 5
    for _ in range(num_hidden):
        w = refs[idx][...]
        b = refs[idx + 1][...]
        idx += 2
        h = _softplus(jnp.dot(h.astype(jnp.bfloat16), w,
                              preferred_element_type=jnp.float32) + b)
    w = refs[idx][...]
    b = refs[idx + 1][...]
    o_ref = refs[idx + 2]
    o_ref[...] = jnp.dot(h.astype(jnp.bfloat16), w,
                         preferred_element_type=jnp.float32) + b


def _conv_call(dstr, srcr, pt, prev, wn, nb, we, sc, sh, *, tpc, tile,
               n_nodes, feat, num_g, off_step, pad_off, coeff):
    two_f = 2 * feat
    n_tiles = dstr.shape[0]
    eim = lambda c, t: (c * tpc + t, 0, 0)
    cim2 = lambda c, t: (0, 0)
    cim3 = lambda c, t: (0, 0, 0)
    edge_specs = [pl.BlockSpec((1, 1, tile), eim) for _ in range(2)]
    edge_specs.append(pl.BlockSpec(pt.shape, cim2))
    if prev[0] is None:
        embx, nucw, nucb = prev[1], prev[2], prev[3]
        body = _conv0_kernel
        extra_in = [embx.astype(jnp.bfloat16), nucw.astype(jnp.bfloat16),
                    nucb.reshape(1, feat)]
        extra_specs = [
            pl.BlockSpec(embx.shape, cim2),
            pl.BlockSpec(nucw.shape, cim2),
            pl.BlockSpec((1, feat), cim2),
        ]
    else:
        part_prev, x_prev, lg, lb = prev
        body = _conv_kernel
        extra_in = [part_prev, x_prev, lg.reshape(1, feat), lb.reshape(1, feat)]
        extra_specs = [
            pl.BlockSpec((2, n_nodes, feat), cim3),
            pl.BlockSpec((1, n_nodes, feat), cim3),
            pl.BlockSpec((1, feat), cim2),
            pl.BlockSpec((1, feat), cim2),
        ]
    e_pad = n_tiles * tile
    cost = pl.CostEstimate(
        flops=int(2 * e_pad * two_f * (2 * n_nodes + we.shape[0] + feat)),
        transcendentals=int(e_pad * (we.shape[0] + 2 * feat)),
        bytes_accessed=int(e_pad * 12 + 4 * n_nodes * (two_f + 4 * feat)))
    return pl.pallas_call(
        functools.partial(body, tpc=tpc, n_nodes=n_nodes, feat=feat,
                          num_g=num_g, off_step=off_step, pad_off=pad_off,
                          coeff=coeff),
        out_shape=(jax.ShapeDtypeStruct((2, n_nodes, feat), jnp.float32),
                   jax.ShapeDtypeStruct((2, n_nodes, feat), jnp.float32)),
        grid=(2, tpc),
        in_specs=edge_specs + extra_specs + [
            pl.BlockSpec((feat, 2 * two_f), cim2),
            pl.BlockSpec((1, 2 * two_f), cim2),
            pl.BlockSpec(we.shape, cim2),
            pl.BlockSpec((1, two_f), cim2),
            pl.BlockSpec((1, two_f), cim2),
        ],
        out_specs=(pl.BlockSpec((1, n_nodes, feat), lambda c, t: (c, 0, 0)),
                   pl.BlockSpec((1, n_nodes, feat), lambda c, t: (c, 0, 0))),
        scratch_shapes=[pltpu.VMEM((2 * n_nodes, two_f + 8), jnp.bfloat16),
                        pltpu.VMEM((2 * n_nodes, two_f + 8), jnp.bfloat16),
                        pltpu.VMEM((2 * n_nodes, two_f + 8), jnp.bfloat16),
                        pltpu.VMEM((n_nodes, feat), jnp.float32)],
        compiler_params=pltpu.CompilerParams(
            dimension_semantics=("parallel", "arbitrary"),
            vmem_limit_bytes=_VMEM_LIMIT),
        cost_estimate=cost,
    )(dstr, srcr, pt, *extra_in, wn.astype(jnp.bfloat16),
      nb.reshape(1, 2 * two_f), we.astype(jnp.bfloat16),
      sc.reshape(1, two_f), sh.reshape(1, two_f))


def _head_call(part_prev, x_prev, lg, lb, pool, hidden, w_out, b_out, *,
               n_nodes, feat):
    n_graphs = pool.shape[0]
    n_targets = w_out.shape[1]
    cim2 = lambda: (0, 0)
    inputs = [part_prev, x_prev, lg.reshape(1, feat), lb.reshape(1, feat),
              pool]
    in_specs = [
        pl.BlockSpec((2, n_nodes, feat), lambda: (0, 0, 0)),
        pl.BlockSpec((1, n_nodes, feat), lambda: (0, 0, 0)),
        pl.BlockSpec((1, feat), cim2),
        pl.BlockSpec((1, feat), cim2),
        pl.BlockSpec(pool.shape, cim2),
    ]
    for w, b in hidden:
        inputs += [w.astype(jnp.bfloat16), b.reshape(1, -1)]
        in_specs += [pl.BlockSpec(w.shape, cim2),
                     pl.BlockSpec((1, b.shape[0]), cim2)]
    inputs += [w_out.astype(jnp.bfloat16), b_out.reshape(1, n_targets)]
    in_specs += [pl.BlockSpec(w_out.shape, cim2),
                 pl.BlockSpec((1, n_targets), cim2)]
    return pl.pallas_call(
        functools.partial(_head_kernel, num_hidden=len(hidden)),
        out_shape=jax.ShapeDtypeStruct((n_graphs, n_targets), jnp.float32),
        in_specs=in_specs,
        out_specs=pl.BlockSpec((n_graphs, n_targets), cim2),
        compiler_params=pltpu.CompilerParams(vmem_limit_bytes=_VMEM_LIMIT),
    )(*inputs)


def kernel(embedding, nuc_w, nuc_b,
           conv0_w_node, conv0_node_b, conv0_w_edge, conv0_scale, conv0_shift, conv0_ln_gamma, conv0_ln_beta,
           conv1_w_node, conv1_node_b, conv1_w_edge, conv1_scale, conv1_shift, conv1_ln_gamma, conv1_ln_beta,
           conv2_w_node, conv2_node_b, conv2_w_edge, conv2_scale, conv2_shift, conv2_ln_gamma, conv2_ln_beta,
           conv3_w_node, conv3_node_b, conv3_w_edge, conv3_scale, conv3_shift, conv3_ln_gamma, conv3_ln_beta,
           conv4_w_node, conv4_node_b, conv4_w_edge, conv4_scale, conv4_shift, conv4_ln_gamma, conv4_ln_beta,
           conv5_w_node, conv5_node_b, conv5_w_edge, conv5_scale, conv5_shift, conv5_ln_gamma, conv5_ln_beta,
           conv_to_fc_w, conv_to_fc_b,
           fc0_w, fc0_b, fc1_w, fc1_b, fc2_w, fc2_b,
           out_w, out_b,
           atomic_numbers, positions, batch, neighbors_index):
    num_graphs = 2
    num_g = 100
    cutoff = 6.0
    n_nodes = atomic_numbers.shape[0]
    feat = nuc_w.shape[1]
    two_f = 2 * feat

    convs = [
        (conv0_w_node, conv0_node_b, conv0_w_edge, conv0_scale, conv0_shift, conv0_ln_gamma, conv0_ln_beta),
        (conv1_w_node, conv1_node_b, conv1_w_edge, conv1_scale, conv1_shift, conv1_ln_gamma, conv1_ln_beta),
        (conv2_w_node, conv2_node_b, conv2_w_edge, conv2_scale, conv2_shift, conv2_ln_gamma, conv2_ln_beta),
        (conv3_w_node, conv3_node_b, conv3_w_edge, conv3_scale, conv3_shift, conv3_ln_gamma, conv3_ln_beta),
        (conv4_w_node, conv4_node_b, conv4_w_edge, conv4_scale, conv4_shift, conv4_ln_gamma, conv4_ln_beta),
        (conv5_w_node, conv5_node_b, conv5_w_edge, conv5_scale, conv5_shift, conv5_ln_gamma, conv5_ln_beta),
    ]

    src = neighbors_index[0]
    dst = neighbors_index[1]
    n_edges = src.shape[0]

    sigma = cutoff / (num_g - 1)
    coeff = -0.5 / float(sigma ** 2)
    off_step = sigma
    pad_off = 1e3
    pt = jnp.zeros((n_nodes, 8), jnp.float32).at[:, :3].set(
        positions.astype(jnp.float32))

    tile = _EDGE_TILE
    n_tiles = _round_up(_round_up(max(n_edges, 1), tile) // tile, 2)
    e_pad = n_tiles * tile
    if e_pad != n_edges:
        p = e_pad - n_edges
        src = jnp.concatenate([src, jnp.zeros((p,), src.dtype)])
        dst = jnp.concatenate([dst, jnp.full((p,), -1, dst.dtype)])
    tpc = n_tiles // 2
    dstr = dst.reshape(n_tiles, 1, tile)
    srcr = (src + n_nodes).reshape(n_tiles, 1, tile)

    embx = embedding[atomic_numbers - 1]

    onehot = (batch[None, :] == jnp.arange(num_graphs,
                                           dtype=batch.dtype)[:, None]
              ).astype(jnp.float32)
    pool = onehot / jnp.maximum(onehot.sum(axis=1, keepdims=True), 1.0)

    prev = (None, embx, nuc_w, nuc_b)
    for k, (wn, nb, we, sc, sh, lg, lb) in enumerate(convs):
        part, xout = _conv_call(
            dstr, srcr, pt, prev, wn, nb, we, sc, sh, tpc=tpc, tile=tile,
            n_nodes=n_nodes, feat=feat, num_g=num_g, off_step=off_step,
            pad_off=pad_off, coeff=coeff)
        prev = (part, xout[:1], lg, lb)

    hidden = [(conv_to_fc_w, conv_to_fc_b), (fc0_w, fc0_b),
              (fc1_w, fc1_b), (fc2_w, fc2_b)]
    part, x_prev, lg, lb = prev
    return _head_call(part, x_prev, lg, lb, pool, hidden, out_w, out_b,
                      n_nodes=n_nodes, feat=feat)


# speedup vs baseline: 12.4552x; 1.0101x over previous
import functools

import jax
import jax.numpy as jnp
from jax import lax
from jax.experimental import pallas as pl
from jax.experimental.pallas import tpu as pltpu

_VMEM_LIMIT = 48 * 1024 * 1024
_EDGE_TILE = 8192


def _round_up(x, m):
    return ((x + m - 1) // m) * m


def _softplus(x):
    return jnp.maximum(x, 0.0) + jnp.log1p(jnp.exp(-jnp.abs(x)))


def _edge_step(dst_ref, src_ref, we_ref, sc_ref, hhi_scr, hmid_scr,
               hlo_scr, acc_ref, *, n_nodes, feat, num_g, off_step, pad_off,
               coeff):
    two_f = 2 * feat
    n2 = 2 * n_nodes
    row2 = lax.broadcasted_iota(jnp.int32, (n2, 1), 0)
    tgt = jnp.where(row2 < n_nodes, dst_ref[0], src_ref[0])
    oh2 = (row2 == tgt).astype(jnp.bfloat16)
    dims = (((0,), (0,)), ((), ()))
    zfull = (lax.dot_general(oh2, hhi_scr[...], dims,
                             preferred_element_type=jnp.float32)
             + lax.dot_general(oh2, hmid_scr[...], dims,
                               preferred_element_type=jnp.float32)
             + lax.dot_general(oh2, hlo_scr[...], dims,
                               preferred_element_type=jnp.float32))
    diff = zfull[:, two_f:]
    sq = diff * diff
    d = jnp.sqrt(jnp.sum(sq, axis=1, keepdims=True))
    gcol = lax.broadcasted_iota(jnp.int32, (1, we_ref.shape[0]), 1
                                ).astype(jnp.float32)
    off = jnp.where(gcol < num_g, gcol * off_step, pad_off)
    u = d - off
    ef = jnp.exp(coeff * (u * u)).astype(jnp.bfloat16)
    z = (zfull[:, :two_f]
         + jnp.dot(ef, we_ref[...], preferred_element_type=jnp.float32)
         * sc_ref[...])
    gate = pl.reciprocal(1.0 + jnp.exp(-z[:, :feat]), approx=True)
    val = _softplus(z[:, feat:])
    msg = (gate * val).astype(jnp.bfloat16)
    acc_ref[...] += jnp.dot(oh2[:n_nodes], msg,
                            preferred_element_type=jnp.float32)


def _finalize(part0, part1, x_prev, gamma, beta):
    a = part0 + part1
    mean = jnp.mean(a, axis=-1, keepdims=True)
    var = jnp.mean((a - mean) ** 2, axis=-1, keepdims=True)
    ln = (a - mean) * lax.rsqrt(var + 1e-5) * gamma + beta
    return _softplus(ln + x_prev)


def _conv0_kernel(dst_ref, src_ref, pt_ref, embx_ref, nucw_ref, nucb_ref,
                  wn_ref, nb_ref, we_ref, sc_ref, sh_ref, part_ref, xout_ref,
                  hhi_scr, hmid_scr, hlo_scr, acc_ref, *, tpc, n_nodes,
                  feat, num_g, off_step, pad_off, coeff):
    t = pl.program_id(1)

    @pl.when(t == 0)
    def _():
        x = jnp.dot(embx_ref[...], nucw_ref[...],
                    preferred_element_type=jnp.float32) + nucb_ref[...]
        xout_ref[0] = x
        h = jnp.dot(x.astype(jnp.bfloat16), wn_ref[...],
                    preferred_element_type=jnp.float32) + nb_ref[...]
        two_f = 2 * feat
        pt = pt_ref[...]
        sc4 = jnp.concatenate([sc_ref[...], sc_ref[...]], axis=1)
        sh4 = jnp.concatenate([sh_ref[...], sh_ref[...]], axis=1) * 0.5
        hs = h * sc4 + sh4
        tab = jnp.concatenate(
            [jnp.concatenate([hs[:, :two_f], pt], axis=1),
             jnp.concatenate([hs[:, two_f:], -pt], axis=1)], axis=0)
        thi = tab.astype(jnp.bfloat16)
        r1 = tab - thi.astype(jnp.float32)
        tmid = r1.astype(jnp.bfloat16)
        hhi_scr[...] = thi
        hmid_scr[...] = tmid
        hlo_scr[...] = (r1 - tmid.astype(jnp.float32)).astype(jnp.bfloat16)
        acc_ref[...] = jnp.zeros_like(acc_ref)

    _edge_step(dst_ref, src_ref, we_ref, sc_ref, hhi_scr, hmid_scr,
               hlo_scr, acc_ref, n_nodes=n_nodes, feat=feat, num_g=num_g,
               off_step=off_step, pad_off=pad_off, coeff=coeff)

    @pl.when(t == tpc - 1)
    def _():
        part_ref[0] = acc_ref[...]


def _conv_kernel(dst_ref, src_ref, pt_ref, pprev_ref, xprev_ref, g_ref, b_ref,
                 wn_ref, nb_ref, we_ref, sc_ref, sh_ref, part_ref, xout_ref,
                 hhi_scr, hmid_scr, hlo_scr, acc_ref, *, tpc, n_nodes,
                 feat, num_g, off_step, pad_off, coeff):
    t = pl.program_id(1)

    @pl.when(t == 0)
    def _():
        x = _finalize(pprev_ref[0], pprev_ref[1], xprev_ref[0],
                      g_ref[...], b_ref[...])
        xout_ref[0] = x
        h = jnp.dot(x.astype(jnp.bfloat16), wn_ref[...],
                    preferred_element_type=jnp.float32) + nb_ref[...]
        two_f = 2 * feat
        pt = pt_ref[...]
        sc4 = jnp.concatenate([sc_ref[...], sc_ref[...]], axis=1)
        sh4 = jnp.concatenate([sh_ref[...], sh_ref[...]], axis=1) * 0.5
        hs = h * sc4 + sh4
        tab = jnp.concatenate(
            [jnp.concatenate([hs[:, :two_f], pt], axis=1),
             jnp.concatenate([hs[:, two_f:], -pt], axis=1)], axis=0)
        thi = tab.astype(jnp.bfloat16)
        r1 = tab - thi.astype(jnp.float32)
        tmid = r1.astype(jnp.bfloat16)
        hhi_scr[...] = thi
        hmid_scr[...] = tmid
        hlo_scr[...] = (r1 - tmid.astype(jnp.float32)).astype(jnp.bfloat16)
        acc_ref[...] = jnp.zeros_like(acc_ref)

    _edge_step(dst_ref, src_ref, we_ref, sc_ref, hhi_scr, hmid_scr,
               hlo_scr, acc_ref, n_nodes=n_nodes, feat=feat, num_g=num_g,
               off_step=off_step, pad_off=pad_off, coeff=coeff)

    @pl.when(t == tpc - 1)
    def _():
        part_ref[0] = acc_ref[...]


def _head_kernel(*refs, num_hidden):
    pprev_ref, xprev_ref, g_ref, b_ref, pool_ref = refs[:5]
    x = _finalize(pprev_ref[0], pprev_ref[1], xprev_ref[0],
                  g_ref[...], b_ref[...])
    h = jnp.dot(pool_ref[...], x, preferred_element_type=jnp.float32)
    idx = 5
    for _ in range(num_hidden):
        w = refs[idx][...]
        b = refs[idx + 1][...]
        idx += 2
        h = _softplus(jnp.dot(h.astype(jnp.bfloat16), w,
                              preferred_element_type=jnp.float32) + b)
    w = refs[idx][...]
    b = refs[idx + 1][...]
    o_ref = refs[idx + 2]
    o_ref[...] = jnp.dot(h.astype(jnp.bfloat16), w,
                         preferred_element_type=jnp.float32) + b


def _conv_call(dstr, srcr, pt, prev, wn, nb, we, sc, sh, *, tpc, tile,
               n_nodes, feat, num_g, off_step, pad_off, coeff):
    two_f = 2 * feat
    n_tiles = dstr.shape[0]
    eim = lambda c, t: (c * tpc + t, 0, 0)
    cim2 = lambda c, t: (0, 0)
    cim3 = lambda c, t: (0, 0, 0)
    edge_specs = [pl.BlockSpec((1, 1, tile), eim) for _ in range(2)]
    edge_specs.append(pl.BlockSpec(pt.shape, cim2))
    if prev[0] is None:
        embx, nucw, nucb = prev[1], prev[2], prev[3]
        body = _conv0_kernel
        extra_in = [embx.astype(jnp.bfloat16), nucw.astype(jnp.bfloat16),
                    nucb.reshape(1, feat)]
        extra_specs = [
            pl.BlockSpec(embx.shape, cim2),
            pl.BlockSpec(nucw.shape, cim2),
            pl.BlockSpec((1, feat), cim2),
        ]
    else:
        part_prev, x_prev, lg, lb = prev
        body = _conv_kernel
        extra_in = [part_prev, x_prev, lg.reshape(1, feat), lb.reshape(1, feat)]
        extra_specs = [
            pl.BlockSpec((2, n_nodes, feat), cim3),
            pl.BlockSpec((1, n_nodes, feat), cim3),
            pl.BlockSpec((1, feat), cim2),
            pl.BlockSpec((1, feat), cim2),
        ]
    e_pad = n_tiles * tile
    cost = pl.CostEstimate(
        flops=int(2 * e_pad * two_f * (2 * n_nodes + we.shape[0] + feat)),
        transcendentals=int(e_pad * (we.shape[0] + 2 * feat)),
        bytes_accessed=int(e_pad * 12 + 4 * n_nodes * (two_f + 4 * feat)))
    return pl.pallas_call(
        functools.partial(body, tpc=tpc, n_nodes=n_nodes, feat=feat,
                          num_g=num_g, off_step=off_step, pad_off=pad_off,
                          coeff=coeff),
        out_shape=(jax.ShapeDtypeStruct((2, n_nodes, feat), jnp.float32),
                   jax.ShapeDtypeStruct((2, n_nodes, feat), jnp.float32)),
        grid=(2, tpc),
        in_specs=edge_specs + extra_specs + [
            pl.BlockSpec((feat, 2 * two_f), cim2),
            pl.BlockSpec((1, 2 * two_f), cim2),
            pl.BlockSpec(we.shape, cim2),
            pl.BlockSpec((1, two_f), cim2),
            pl.BlockSpec((1, two_f), cim2),
        ],
        out_specs=(pl.BlockSpec((1, n_nodes, feat), lambda c, t: (c, 0, 0)),
                   pl.BlockSpec((1, n_nodes, feat), lambda c, t: (c, 0, 0))),
        scratch_shapes=[pltpu.VMEM((2 * n_nodes, two_f + 8), jnp.bfloat16),
                        pltpu.VMEM((2 * n_nodes, two_f + 8), jnp.bfloat16),
                        pltpu.VMEM((2 * n_nodes, two_f + 8), jnp.bfloat16),
                        pltpu.VMEM((n_nodes, feat), jnp.float32)],
        compiler_params=pltpu.CompilerParams(
            dimension_semantics=("parallel", "arbitrary"),
            vmem_limit_bytes=_VMEM_LIMIT),
        cost_estimate=cost,
    )(dstr, srcr, pt, *extra_in, wn.astype(jnp.bfloat16),
      nb.reshape(1, 2 * two_f), we.astype(jnp.bfloat16),
      sc.reshape(1, two_f), sh.reshape(1, two_f))


def _head_call(part_prev, x_prev, lg, lb, pool, hidden, w_out, b_out, *,
               n_nodes, feat):
    n_graphs = pool.shape[0]
    n_targets = w_out.shape[1]
    cim2 = lambda: (0, 0)
    inputs = [part_prev, x_prev, lg.reshape(1, feat), lb.reshape(1, feat),
              pool]
    in_specs = [
        pl.BlockSpec((2, n_nodes, feat), lambda: (0, 0, 0)),
        pl.BlockSpec((1, n_nodes, feat), lambda: (0, 0, 0)),
        pl.BlockSpec((1, feat), cim2),
        pl.BlockSpec((1, feat), cim2),
        pl.BlockSpec(pool.shape, cim2),
    ]
    for w, b in hidden:
        inputs += [w.astype(jnp.bfloat16), b.reshape(1, -1)]
        in_specs += [pl.BlockSpec(w.shape, cim2),
                     pl.BlockSpec((1, b.shape[0]), cim2)]
    inputs += [w_out.astype(jnp.bfloat16), b_out.reshape(1, n_targets)]
    in_specs += [pl.BlockSpec(w_out.shape, cim2),
                 pl.BlockSpec((1, n_targets), cim2)]
    return pl.pallas_call(
        functools.partial(_head_kernel, num_hidden=len(hidden)),
        out_shape=jax.ShapeDtypeStruct((n_graphs, n_targets), jnp.float32),
        in_specs=in_specs,
        out_specs=pl.BlockSpec((n_graphs, n_targets), cim2),
        compiler_params=pltpu.CompilerParams(vmem_limit_bytes=_VMEM_LIMIT),
    )(*inputs)


def kernel(embedding, nuc_w, nuc_b,
           conv0_w_node, conv0_node_b, conv0_w_edge, conv0_scale, conv0_shift, conv0_ln_gamma, conv0_ln_beta,
           conv1_w_node, conv1_node_b, conv1_w_edge, conv1_scale, conv1_shift, conv1_ln_gamma, conv1_ln_beta,
           conv2_w_node, conv2_node_b, conv2_w_edge, conv2_scale, conv2_shift, conv2_ln_gamma, conv2_ln_beta,
           conv3_w_node, conv3_node_b, conv3_w_edge, conv3_scale, conv3_shift, conv3_ln_gamma, conv3_ln_beta,
           conv4_w_node, conv4_node_b, conv4_w_edge, conv4_scale, conv4_shift, conv4_ln_gamma, conv4_ln_beta,
           conv5_w_node, conv5_node_b, conv5_w_edge, conv5_scale, conv5_shift, conv5_ln_gamma, conv5_ln_beta,
           conv_to_fc_w, conv_to_fc_b,
           fc0_w, fc0_b, fc1_w, fc1_b, fc2_w, fc2_b,
           out_w, out_b,
           atomic_numbers, positions, batch, neighbors_index):
    num_graphs = 2
    num_g = 100
    cutoff = 6.0
    n_nodes = atomic_numbers.shape[0]
    feat = nuc_w.shape[1]
    two_f = 2 * feat

    convs = [
        (conv0_w_node, conv0_node_b, conv0_w_edge, conv0_scale, conv0_shift, conv0_ln_gamma, conv0_ln_beta),
        (conv1_w_node, conv1_node_b, conv1_w_edge, conv1_scale, conv1_shift, conv1_ln_gamma, conv1_ln_beta),
        (conv2_w_node, conv2_node_b, conv2_w_edge, conv2_scale, conv2_shift, conv2_ln_gamma, conv2_ln_beta),
        (conv3_w_node, conv3_node_b, conv3_w_edge, conv3_scale, conv3_shift, conv3_ln_gamma, conv3_ln_beta),
        (conv4_w_node, conv4_node_b, conv4_w_edge, conv4_scale, conv4_shift, conv4_ln_gamma, conv4_ln_beta),
        (conv5_w_node, conv5_node_b, conv5_w_edge, conv5_scale, conv5_shift, conv5_ln_gamma, conv5_ln_beta),
    ]

    src = neighbors_index[0]
    dst = neighbors_index[1]
    n_edges = src.shape[0]

    sigma = cutoff / (num_g - 1)
    coeff = -0.5 / float(sigma ** 2)
    off_step = sigma
    pad_off = 1e3
    pt = jnp.zeros((n_nodes, 8), jnp.float32).at[:, :3].set(
        positions.astype(jnp.float32))

    tile = _EDGE_TILE
    n_tiles = _round_up(_round_up(max(n_edges, 1), tile) // tile, 2)
    e_pad = n_tiles * tile
    if e_pad != n_edges:
        p = e_pad - n_edges
        src = jnp.concatenate([src, jnp.zeros((p,), src.dtype)])
        dst = jnp.concatenate([dst, jnp.full((p,), -1, dst.dtype)])
    tpc = n_tiles // 2
    dstr = dst.reshape(n_tiles, 1, tile)
    srcr = (src + n_nodes).reshape(n_tiles, 1, tile)

    embx = embedding[atomic_numbers - 1]

    onehot = (batch[None, :] == jnp.arange(num_graphs,
                                           dtype=batch.dtype)[:, None]
              ).astype(jnp.float32)
    pool = onehot / jnp.maximum(onehot.sum(axis=1, keepdims=True), 1.0)

    prev = (None, embx, nuc_w, nuc_b)
    for k, (wn, nb, we, sc, sh, lg, lb) in enumerate(convs):
        part, xout = _conv_call(
            dstr, srcr, pt, prev, wn, nb, we, sc, sh, tpc=tpc, tile=tile,
            n_nodes=n_nodes, feat=feat, num_g=num_g, off_step=off_step,
            pad_off=pad_off, coeff=coeff)
        prev = (part, xout[:1], lg, lb)

    hidden = [(conv_to_fc_w, conv_to_fc_b), (fc0_w, fc0_b),
              (fc1_w, fc1_b), (fc2_w, fc2_b)]
    part, x_prev, lg, lb = prev
    return _head_call(part, x_prev, lg, lb, pool, hidden, out_w, out_b,
                      n_nodes=n_nodes, feat=feat)
